# baseline (speedup 1.0000x reference)
import functools

import jax
import jax.numpy as jnp
from jax import lax
from jax.experimental import pallas as pl
from jax.experimental.pallas import tpu as pltpu
from jax.experimental.pallas import tpu_sc as plsc


def _build_gather(num_rows: int, hidden: int):
    info = plsc.get_sparse_core_info()
    nc, ns = info.num_cores, info.num_subcores
    nw = nc * ns
    assert num_rows % (8 * nw) == 0
    b_per_w = num_rows // nw

    mesh = plsc.VectorSubcoreMesh(core_axis_name="c", subcore_axis_name="s")

    nch = 4
    ch = b_per_w // nch

    @functools.partial(
        pl.kernel,
        mesh=mesh,
        out_type=jax.ShapeDtypeStruct((num_rows, hidden), jnp.float32),
        scratch_types=[
            pltpu.VMEM((b_per_w,), jnp.int32),
            pltpu.VMEM((b_per_w, hidden), jnp.float32),
        ]
        + [pltpu.SemaphoreType.DMA] * (nch + 1),
    )
    def gather_kernel(idx_hbm, table_hbm, out_hbm, idx_v, rows_v, *sems):
        gsems, ssem = sems[:nch], sems[nch]
        wid = lax.axis_index("s") * nc + lax.axis_index("c")
        base = wid * b_per_w
        pltpu.sync_copy(idx_hbm.at[pl.ds(base, b_per_w)], idx_v)
        gathers = [
            pltpu.async_copy(
                table_hbm.at[idx_v.at[pl.ds(c * ch, ch)]],
                rows_v.at[pl.ds(c * ch, ch)],
                gsems[c],
            )
            for c in range(nch)
        ]
        scatters = []
        for c in range(nch):
            gathers[c].wait()
            scatters.append(
                pltpu.async_copy(
                    rows_v.at[pl.ds(c * ch, ch)],
                    out_hbm.at[pl.ds(base + c * ch, ch)],
                    ssem,
                )
            )
        for s in scatters:
            s.wait()

    return gather_kernel


def kernel(position_ids, table):
    num_rows = position_ids.shape[-1]
    hidden = table.shape[-1]
    ids_flat = position_ids.reshape(num_rows).astype(jnp.int32)
    out = _build_gather(num_rows, hidden)(ids_flat, table)
    return out.reshape(num_rows, 1, hidden)


# speedup vs baseline: 1.3931x; 1.3931x over previous
import functools

import jax
import jax.numpy as jnp
from jax import lax
from jax.experimental import pallas as pl
from jax.experimental.pallas import tpu as pltpu
from jax.experimental.pallas import tpu_sc as plsc


def _build_gather(num_batch: int, num_rows: int, hidden: int):
    info = plsc.get_sparse_core_info()
    nc, ns = info.num_cores, info.num_subcores
    nw = nc * ns
    assert num_rows % (8 * nw) == 0
    b_per_w = num_rows // nw

    mesh = plsc.VectorSubcoreMesh(core_axis_name="c", subcore_axis_name="s")

    nch = 4
    ch = b_per_w // nch

    @functools.partial(
        pl.kernel,
        mesh=mesh,
        out_type=jax.ShapeDtypeStruct((num_rows, num_batch, hidden), jnp.float32),
        scratch_types=[
            pltpu.VMEM((b_per_w,), jnp.int32),
            pltpu.VMEM((b_per_w, hidden), jnp.float32),
        ]
        + [pltpu.SemaphoreType.DMA] * (nch + 1),
    )
    def gather_kernel(idx_hbm, table_hbm, out_hbm, idx_v, rows_v, *sems):
        gsems, ssem = sems[:nch], sems[nch]
        wid = lax.axis_index("s") * nc + lax.axis_index("c")
        base = wid * b_per_w
        pltpu.sync_copy(idx_hbm.at[0, pl.ds(base, b_per_w)], idx_v)
        gathers = [
            pltpu.async_copy(
                table_hbm.at[idx_v.at[pl.ds(c * ch, ch)]],
                rows_v.at[pl.ds(c * ch, ch)],
                gsems[c],
            )
            for c in range(nch)
        ]
        scatters = []
        for c in range(nch):
            gathers[c].wait()
            scatters.append(
                pltpu.async_copy(
                    rows_v.at[pl.ds(c * ch, ch)],
                    out_hbm.at[pl.ds(base + c * ch, ch), 0],
                    ssem,
                )
            )
        for s in scatters:
            s.wait()

    return gather_kernel


def kernel(position_ids, table):
    num_batch, num_rows = position_ids.shape
    hidden = table.shape[-1]
    ids = position_ids.astype(jnp.int32)
    return _build_gather(num_batch, num_rows, hidden)(ids, table)


# speedup vs baseline: 1.4311x; 1.0273x over previous
import functools

import jax
import jax.numpy as jnp
from jax import lax
from jax.experimental import pallas as pl
from jax.experimental.pallas import tpu as pltpu
from jax.experimental.pallas import tpu_sc as plsc


def _build_gather(num_batch: int, num_rows: int, hidden: int):
    info = plsc.get_sparse_core_info()
    nc, ns = info.num_cores, info.num_subcores
    nw = nc * ns
    assert num_rows % (8 * nw) == 0
    b_per_w = num_rows // nw

    mesh = plsc.VectorSubcoreMesh(core_axis_name="c", subcore_axis_name="s")

    @functools.partial(
        pl.kernel,
        mesh=mesh,
        out_type=jax.ShapeDtypeStruct((num_rows, num_batch, hidden), jnp.float32),
        scratch_types=[
            pltpu.VMEM((b_per_w,), jnp.int32),
            pltpu.VMEM((b_per_w, hidden), jnp.float32),
            pltpu.SemaphoreType.DMA,
        ],
    )
    def gather_kernel(idx_hbm, table_hbm, out_hbm, idx_v, rows_v, sem):
        wid = lax.axis_index("s") * nc + lax.axis_index("c")
        base = wid * b_per_w
        pltpu.sync_copy(idx_hbm.at[0, pl.ds(base, b_per_w)], idx_v)
        pltpu.async_copy(table_hbm.at[idx_v], rows_v, sem).wait()
        pltpu.sync_copy(rows_v, out_hbm.at[pl.ds(base, b_per_w), 0])

    return gather_kernel


def kernel(position_ids, table):
    num_batch, num_rows = position_ids.shape
    hidden = table.shape[-1]
    ids = position_ids.astype(jnp.int32)
    return _build_gather(num_batch, num_rows, hidden)(ids, table)
